# Initial kernel scaffold; baseline (speedup 1.0000x reference)
import functools

import jax
import jax.numpy as jnp
from jax import lax
from jax.experimental import pallas as pl
from jax.experimental.pallas import tpu as pltpu
from jax.experimental.pallas import tpu_sc as plsc

N = 10000
E = 320000
F_IN = 128
H1 = 128
H2 = 64

NC = 2
NS = 16
NW = NC * NS
EPW = E // NW
K = 128
NFULL = EPW // K
REM = EPW - NFULL * K
RPS = N // NS
DEGW = 16

_MESH = dict(core_axis_name="c", subcore_axis_name="s")


@functools.partial(
    pl.kernel,
    out_type=jax.ShapeDtypeStruct((NC, N, DEGW), jnp.float32),
    mesh=plsc.VectorSubcoreMesh(**_MESH),
    scratch_types=[
        pltpu.VMEM((K,), jnp.int32),
        pltpu.VMEM((REM,), jnp.int32),
        pltpu.VMEM((K, DEGW), jnp.float32),
        pltpu.VMEM_SHARED((N, DEGW), jnp.float32),
    ],
)
def _deg_kernel(dst_hbm, ones_hbm, zeros_hbm, out_hbm, idx_v, idxr_v, ones_v,
                accum):
    c = lax.axis_index("c")
    s = lax.axis_index("s")
    base = (c * NS + s) * EPW
    pltpu.sync_copy(zeros_hbm, accum.at[pl.ds(s * RPS, RPS)])
    pltpu.sync_copy(ones_hbm, ones_v)
    plsc.subcore_barrier()

    def body(i, carry):
        pltpu.sync_copy(dst_hbm.at[pl.ds(base + i * K, K)], idx_v)
        pltpu.sync_copy(ones_v, accum.at[idx_v], add=True)
        return carry

    lax.fori_loop(0, NFULL, body, 0)
    pltpu.sync_copy(dst_hbm.at[pl.ds(base + NFULL * K, REM)], idxr_v)
    pltpu.sync_copy(ones_v.at[pl.ds(0, REM)], accum.at[idxr_v], add=True)
    plsc.subcore_barrier()
    pltpu.sync_copy(accum.at[pl.ds(s * RPS, RPS)],
                    out_hbm.at[c, pl.ds(s * RPS, RPS)])


def _make_agg(D):
    @functools.partial(
        pl.kernel,
        out_type=jax.ShapeDtypeStruct((NC, N, D), jnp.float32),
        mesh=plsc.VectorSubcoreMesh(**_MESH),
        scratch_types=[
            pltpu.VMEM((K,), jnp.int32),
            pltpu.VMEM((K,), jnp.int32),
            pltpu.VMEM((REM,), jnp.int32),
            pltpu.VMEM((REM,), jnp.int32),
            pltpu.VMEM((K, D), jnp.float32),
            pltpu.VMEM((REM, D), jnp.float32),
            pltpu.VMEM_SHARED((N, D), jnp.float32),
            pltpu.SemaphoreType.DMA,
        ],
    )
    def agg(src_hbm, dst_hbm, table_hbm, zeros_hbm, out_hbm, src_v, dst_v,
            srcr_v, dstr_v, rows_v, rowsr_v, accum, sem):
        c = lax.axis_index("c")
        s = lax.axis_index("s")
        base = (c * NS + s) * EPW
        rows = pl.ds(s * RPS, RPS)

        @pl.when(c == 0)
        def _():
            pltpu.sync_copy(table_hbm.at[rows], accum.at[rows])

        @pl.when(c != 0)
        def _():
            pltpu.sync_copy(zeros_hbm, accum.at[rows])

        plsc.subcore_barrier()

        def body(i, carry):
            off = base + i * K
            pltpu.sync_copy(src_hbm.at[pl.ds(off, K)], src_v)
            pltpu.sync_copy(dst_hbm.at[pl.ds(off, K)], dst_v)
            pltpu.async_copy(table_hbm.at[src_v], rows_v, sem).wait()
            pltpu.sync_copy(rows_v, accum.at[dst_v], add=True)
            return carry

        lax.fori_loop(0, NFULL, body, 0)
        off = base + NFULL * K
        pltpu.sync_copy(src_hbm.at[pl.ds(off, REM)], srcr_v)
        pltpu.sync_copy(dst_hbm.at[pl.ds(off, REM)], dstr_v)
        pltpu.async_copy(table_hbm.at[srcr_v], rowsr_v, sem).wait()
        pltpu.sync_copy(rowsr_v, accum.at[dstr_v], add=True)
        plsc.subcore_barrier()
        pltpu.sync_copy(accum.at[rows], out_hbm.at[c, rows])

    return agg


_agg128 = _make_agg(H1)
_agg64 = _make_agg(H2)


_BM = 2500


def _prep_body(x_ref, w1_ref, d0_ref, d1_ref, table_ref, dinv_ref):
    deg = d0_ref[:, 0] + d1_ref[:, 0] + 1.0
    dinv = lax.rsqrt(deg)
    xw = jnp.dot(x_ref[...], w1_ref[...], preferred_element_type=jnp.float32)
    table_ref[...] = xw * dinv[:, None]
    dinv_ref[...] = dinv


_prep = pl.pallas_call(
    _prep_body,
    grid=(N // _BM,),
    in_specs=[
        pl.BlockSpec((_BM, F_IN), lambda i: (i, 0)),
        pl.BlockSpec((F_IN, H1), lambda i: (0, 0)),
        pl.BlockSpec((_BM, DEGW), lambda i: (i, 0)),
        pl.BlockSpec((_BM, DEGW), lambda i: (i, 0)),
    ],
    out_specs=[
        pl.BlockSpec((_BM, H1), lambda i: (i, 0)),
        pl.BlockSpec((_BM,), lambda i: (i,)),
    ],
    out_shape=[
        jax.ShapeDtypeStruct((N, H1), jnp.float32),
        jax.ShapeDtypeStruct((N,), jnp.float32),
    ],
)


def _mid_body(p0_ref, p1_ref, dinv_ref, b1_ref, w2_ref, t2_ref):
    dinv = dinv_ref[...]
    h = jnp.maximum((p0_ref[...] + p1_ref[...]) * dinv[:, None]
                    + b1_ref[...][None, :], 0.0)
    t2 = jnp.dot(h, w2_ref[...], preferred_element_type=jnp.float32)
    t2_ref[...] = t2 * dinv[:, None]


_mid = pl.pallas_call(
    _mid_body,
    grid=(N // _BM,),
    in_specs=[
        pl.BlockSpec((_BM, H1), lambda i: (i, 0)),
        pl.BlockSpec((_BM, H1), lambda i: (i, 0)),
        pl.BlockSpec((_BM,), lambda i: (i,)),
        pl.BlockSpec((H1,), lambda i: (0,)),
        pl.BlockSpec((H1, H2), lambda i: (0, 0)),
    ],
    out_specs=pl.BlockSpec((_BM, H2), lambda i: (i, 0)),
    out_shape=jax.ShapeDtypeStruct((N, H2), jnp.float32),
)


def _fin_body(q0_ref, q1_ref, dinv_ref, b2_ref, out_ref):
    out_ref[...] = ((q0_ref[...] + q1_ref[...]) * dinv_ref[...][:, None]
                    + b2_ref[...][None, :])


_fin = pl.pallas_call(
    _fin_body,
    grid=(N // _BM,),
    in_specs=[
        pl.BlockSpec((_BM, H2), lambda i: (i, 0)),
        pl.BlockSpec((_BM, H2), lambda i: (i, 0)),
        pl.BlockSpec((_BM,), lambda i: (i,)),
        pl.BlockSpec((H2,), lambda i: (0,)),
    ],
    out_specs=pl.BlockSpec((_BM, H2), lambda i: (i, 0)),
    out_shape=jax.ShapeDtypeStruct((N, H2), jnp.float32),
)


def kernel(x, edge_index, W1, b1, W2, b2):
    src = edge_index[0]
    dst = edge_index[1]
    ones_deg = jnp.ones((K, DEGW), jnp.float32)
    zeros_deg = jnp.zeros((RPS, DEGW), jnp.float32)
    zeros_h1 = jnp.zeros((RPS, H1), jnp.float32)
    zeros_h2 = jnp.zeros((RPS, H2), jnp.float32)

    deg = _deg_kernel(dst, ones_deg, zeros_deg)
    table1, dinv = _prep(x, W1, deg[0], deg[1])
    p = _agg128(src, dst, table1, zeros_h1)
    table2 = _mid(p[0], p[1], dinv, b1, W2)
    q = _agg64(src, dst, table2, zeros_h2)
    return _fin(q[0], q[1], dinv, b2)


# speedup vs baseline: 17.3233x; 17.3233x over previous
import functools

import jax
import jax.numpy as jnp
from jax import lax
from jax.experimental import pallas as pl
from jax.experimental.pallas import tpu as pltpu
from jax.experimental.pallas import tpu_sc as plsc

N = 10000
E = 320000
F_IN = 128
H1 = 128
H2 = 64

NC = 2
NS = 16
NW = NC * NS
EPW = E // NW
K = 128
NFULL = EPW // K
REM = EPW - NFULL * K
RPS_A = 632
RPS_LAST = N - (NS - 1) * RPS_A
DEGW = 16

_MESH = dict(core_axis_name="c", subcore_axis_name="s")


def _per_rows(s, fn):
    @pl.when(s < NS - 1)
    def _():
        fn(s * RPS_A, RPS_A)

    @pl.when(s == NS - 1)
    def _():
        fn((NS - 1) * RPS_A, RPS_LAST)


@functools.partial(
    pl.kernel,
    out_type=jax.ShapeDtypeStruct((NC, N, DEGW), jnp.float32),
    mesh=plsc.VectorSubcoreMesh(**_MESH),
    compiler_params=pltpu.CompilerParams(use_tc_tiling_on_sc=False),
    scratch_types=[
        pltpu.VMEM((K,), jnp.int32),
        pltpu.VMEM((REM,), jnp.int32),
        pltpu.VMEM((K, DEGW), jnp.float32),
        pltpu.VMEM_SHARED((N, DEGW), jnp.float32),
    ],
)
def _deg_kernel(dst_hbm, ones_hbm, zeros_hbm, out_hbm, idx_v, idxr_v, ones_v,
                accum):
    c = lax.axis_index("c")
    s = lax.axis_index("s")
    base = (c * NS + s) * EPW
    _per_rows(s, lambda r0, n: pltpu.sync_copy(
        zeros_hbm.at[pl.ds(0, n)], accum.at[pl.ds(r0, n)]))
    pltpu.sync_copy(ones_hbm, ones_v)
    plsc.subcore_barrier()

    def body(i, carry):
        pltpu.sync_copy(dst_hbm.at[pl.ds(base + i * K, K)], idx_v)
        pltpu.sync_copy(ones_v, accum.at[idx_v], add=True)
        return carry

    lax.fori_loop(0, NFULL, body, 0)
    pltpu.sync_copy(dst_hbm.at[pl.ds(base + NFULL * K, REM)], idxr_v)
    pltpu.sync_copy(ones_v.at[pl.ds(0, REM)], accum.at[idxr_v], add=True)
    plsc.subcore_barrier()
    _per_rows(s, lambda r0, n: pltpu.sync_copy(
        accum.at[pl.ds(r0, n)], out_hbm.at[c, pl.ds(r0, n)]))


def _make_agg(D):
    @functools.partial(
        pl.kernel,
        out_type=jax.ShapeDtypeStruct((NC, N, D), jnp.float32),
        mesh=plsc.VectorSubcoreMesh(**_MESH),
        compiler_params=pltpu.CompilerParams(use_tc_tiling_on_sc=False),
        scratch_types=[
            pltpu.VMEM((K,), jnp.int32),
            pltpu.VMEM((K,), jnp.int32),
            pltpu.VMEM((REM,), jnp.int32),
            pltpu.VMEM((REM,), jnp.int32),
            pltpu.VMEM((K, D), jnp.float32),
            pltpu.VMEM((REM, D), jnp.float32),
            pltpu.VMEM_SHARED((N, D), jnp.float32),
            pltpu.SemaphoreType.DMA,
        ],
    )
    def agg(src_hbm, dst_hbm, table_hbm, zeros_hbm, out_hbm, src_v, dst_v,
            srcr_v, dstr_v, rows_v, rowsr_v, accum, sem):
        c = lax.axis_index("c")
        s = lax.axis_index("s")
        base = (c * NS + s) * EPW

        @pl.when(c == 0)
        def _():
            _per_rows(s, lambda r0, n: pltpu.sync_copy(
                table_hbm.at[pl.ds(r0, n)], accum.at[pl.ds(r0, n)]))

        @pl.when(c != 0)
        def _():
            _per_rows(s, lambda r0, n: pltpu.sync_copy(
                zeros_hbm.at[pl.ds(0, n)], accum.at[pl.ds(r0, n)]))

        plsc.subcore_barrier()

        def body(i, carry):
            off = base + i * K
            pltpu.sync_copy(src_hbm.at[pl.ds(off, K)], src_v)
            pltpu.sync_copy(dst_hbm.at[pl.ds(off, K)], dst_v)
            pltpu.async_copy(table_hbm.at[src_v], rows_v, sem).wait()
            pltpu.sync_copy(rows_v, accum.at[dst_v], add=True)
            return carry

        lax.fori_loop(0, NFULL, body, 0)
        off = base + NFULL * K
        pltpu.sync_copy(src_hbm.at[pl.ds(off, REM)], srcr_v)
        pltpu.sync_copy(dst_hbm.at[pl.ds(off, REM)], dstr_v)
        pltpu.async_copy(table_hbm.at[srcr_v], rowsr_v, sem).wait()
        pltpu.sync_copy(rowsr_v, accum.at[dstr_v], add=True)
        plsc.subcore_barrier()
        _per_rows(s, lambda r0, n: pltpu.sync_copy(
            accum.at[pl.ds(r0, n)], out_hbm.at[c, pl.ds(r0, n)]))

    return agg


_agg128 = _make_agg(H1)
_agg64 = _make_agg(H2)


_BM = 2000


def _prep_body(x_ref, w1_ref, d0_ref, d1_ref, table_ref, dinv_ref):
    deg = d0_ref[:, 0] + d1_ref[:, 0] + 1.0
    dinv = lax.rsqrt(deg)
    xw = jnp.dot(x_ref[...], w1_ref[...], preferred_element_type=jnp.float32)
    table_ref[...] = xw * dinv[:, None]
    dinv_ref[...] = dinv[:, None]


_prep = pl.pallas_call(
    _prep_body,
    grid=(N // _BM,),
    in_specs=[
        pl.BlockSpec((_BM, F_IN), lambda i: (i, 0)),
        pl.BlockSpec((F_IN, H1), lambda i: (0, 0)),
        pl.BlockSpec((_BM, DEGW), lambda i: (i, 0)),
        pl.BlockSpec((_BM, DEGW), lambda i: (i, 0)),
    ],
    out_specs=[
        pl.BlockSpec((_BM, H1), lambda i: (i, 0)),
        pl.BlockSpec((_BM, 1), lambda i: (i, 0)),
    ],
    out_shape=[
        jax.ShapeDtypeStruct((N, H1), jnp.float32),
        jax.ShapeDtypeStruct((N, 1), jnp.float32),
    ],
)


def _mid_body(p0_ref, p1_ref, dinv_ref, b1_ref, w2_ref, t2_ref):
    dinv = dinv_ref[...]
    h = jnp.maximum((p0_ref[...] + p1_ref[...]) * dinv
                    + b1_ref[...][None, :], 0.0)
    t2 = jnp.dot(h, w2_ref[...], preferred_element_type=jnp.float32)
    t2_ref[...] = t2 * dinv


_mid = pl.pallas_call(
    _mid_body,
    grid=(N // _BM,),
    in_specs=[
        pl.BlockSpec((_BM, H1), lambda i: (i, 0)),
        pl.BlockSpec((_BM, H1), lambda i: (i, 0)),
        pl.BlockSpec((_BM, 1), lambda i: (i, 0)),
        pl.BlockSpec((H1,), lambda i: (0,)),
        pl.BlockSpec((H1, H2), lambda i: (0, 0)),
    ],
    out_specs=pl.BlockSpec((_BM, H2), lambda i: (i, 0)),
    out_shape=jax.ShapeDtypeStruct((N, H2), jnp.float32),
)


def _fin_body(q0_ref, q1_ref, dinv_ref, b2_ref, out_ref):
    out_ref[...] = ((q0_ref[...] + q1_ref[...]) * dinv_ref[...]
                    + b2_ref[...][None, :])


_fin = pl.pallas_call(
    _fin_body,
    grid=(N // _BM,),
    in_specs=[
        pl.BlockSpec((_BM, H2), lambda i: (i, 0)),
        pl.BlockSpec((_BM, H2), lambda i: (i, 0)),
        pl.BlockSpec((_BM, 1), lambda i: (i, 0)),
        pl.BlockSpec((H2,), lambda i: (0,)),
    ],
    out_specs=pl.BlockSpec((_BM, H2), lambda i: (i, 0)),
    out_shape=jax.ShapeDtypeStruct((N, H2), jnp.float32),
)


def kernel(x, edge_index, W1, b1, W2, b2):
    src = edge_index[0]
    dst = edge_index[1]
    ones_deg = jnp.ones((K, DEGW), jnp.float32)
    zeros_deg = jnp.zeros((RPS_A, DEGW), jnp.float32)
    zeros_h1 = jnp.zeros((RPS_A, H1), jnp.float32)
    zeros_h2 = jnp.zeros((RPS_A, H2), jnp.float32)

    deg = _deg_kernel(dst, ones_deg, zeros_deg)
    table1, dinv = _prep(x, W1, deg[0], deg[1])
    p = _agg128(src, dst, table1, zeros_h1)
    table2 = _mid(p[0], p[1], dinv, b1, W2)
    q = _agg64(src, dst, table2, zeros_h2)
    return _fin(q[0], q[1], dinv, b2)


# speedup vs baseline: 21.9932x; 1.2696x over previous
import functools

import jax
import jax.numpy as jnp
from jax import lax
from jax.experimental import pallas as pl
from jax.experimental.pallas import tpu as pltpu
from jax.experimental.pallas import tpu_sc as plsc

N = 10000
E = 320000
F_IN = 128
H1 = 128
H2 = 64

NC = 2
NS = 16
NW = NC * NS
EPW = E // NW
K = 128
NFULL = EPW // K
REM = EPW - NFULL * K
RPS_A = 632
RPS_LAST = N - (NS - 1) * RPS_A
DEGW = 16

_MESH = dict(core_axis_name="c", subcore_axis_name="s")


def _per_rows(s, fn):
    @pl.when(s < NS - 1)
    def _():
        fn(s * RPS_A, RPS_A)

    @pl.when(s == NS - 1)
    def _():
        fn((NS - 1) * RPS_A, RPS_LAST)


@functools.partial(
    pl.kernel,
    out_type=jax.ShapeDtypeStruct((NC, N, DEGW), jnp.float32),
    mesh=plsc.VectorSubcoreMesh(**_MESH),
    compiler_params=pltpu.CompilerParams(use_tc_tiling_on_sc=False),
    scratch_types=[
        pltpu.VMEM((K,), jnp.int32),
        pltpu.VMEM((K,), jnp.int32),
        pltpu.VMEM((REM,), jnp.int32),
        pltpu.VMEM((K, DEGW), jnp.float32),
        pltpu.VMEM_SHARED((N, DEGW), jnp.float32),
        pltpu.SemaphoreType.DMA,
        pltpu.SemaphoreType.DMA,
    ],
)
def _deg_kernel(dst_hbm, ones_hbm, zeros_hbm, out_hbm, idx_a, idx_b, idxr_v,
                ones_v, accum, sem_a, sem_b):
    c = lax.axis_index("c")
    s = lax.axis_index("s")
    base = (c * NS + s) * EPW
    _per_rows(s, lambda r0, n: pltpu.sync_copy(
        zeros_hbm.at[pl.ds(0, n)], accum.at[pl.ds(r0, n)]))
    pltpu.sync_copy(ones_hbm, ones_v)
    plsc.subcore_barrier()

    def body(j, carry):
        off_a = base + (2 * j) * K
        pltpu.sync_copy(dst_hbm.at[pl.ds(off_a, K)], idx_a)
        sa = pltpu.async_copy(ones_v, accum.at[idx_a], sem_a, add=True)
        pltpu.sync_copy(dst_hbm.at[pl.ds(off_a + K, K)], idx_b)
        sb = pltpu.async_copy(ones_v, accum.at[idx_b], sem_b, add=True)
        sa.wait()
        sb.wait()
        return carry

    lax.fori_loop(0, NFULL // 2, body, 0)
    pltpu.sync_copy(dst_hbm.at[pl.ds(base + NFULL * K, REM)], idxr_v)
    pltpu.sync_copy(ones_v.at[pl.ds(0, REM)], accum.at[idxr_v], add=True)
    plsc.subcore_barrier()
    _per_rows(s, lambda r0, n: pltpu.sync_copy(
        accum.at[pl.ds(r0, n)], out_hbm.at[c, pl.ds(r0, n)]))


def _make_agg(D):
    @functools.partial(
        pl.kernel,
        out_type=jax.ShapeDtypeStruct((NC, N, D), jnp.float32),
        mesh=plsc.VectorSubcoreMesh(**_MESH),
        compiler_params=pltpu.CompilerParams(use_tc_tiling_on_sc=False),
        scratch_types=[
            pltpu.VMEM((K,), jnp.int32),
            pltpu.VMEM((K,), jnp.int32),
            pltpu.VMEM((K,), jnp.int32),
            pltpu.VMEM((K,), jnp.int32),
            pltpu.VMEM((REM,), jnp.int32),
            pltpu.VMEM((REM,), jnp.int32),
            pltpu.VMEM((K, D), jnp.float32),
            pltpu.VMEM((K, D), jnp.float32),
            pltpu.VMEM((REM, D), jnp.float32),
            pltpu.VMEM_SHARED((N, D), jnp.float32),
            pltpu.SemaphoreType.DMA,
            pltpu.SemaphoreType.DMA,
            pltpu.SemaphoreType.DMA,
            pltpu.SemaphoreType.DMA,
        ],
    )
    def agg(src_hbm, dst_hbm, table_hbm, zeros_hbm, out_hbm, src_a, dst_a,
            src_b, dst_b, srcr_v, dstr_v, rows_a, rows_b, rowsr_v,
            accum, gsem_a, gsem_b, ssem_a, ssem_b):
        c = lax.axis_index("c")
        s = lax.axis_index("s")
        base = (c * NS + s) * EPW

        @pl.when(c == 0)
        def _():
            _per_rows(s, lambda r0, n: pltpu.sync_copy(
                table_hbm.at[pl.ds(r0, n)], accum.at[pl.ds(r0, n)]))

        @pl.when(c != 0)
        def _():
            _per_rows(s, lambda r0, n: pltpu.sync_copy(
                zeros_hbm.at[pl.ds(0, n)], accum.at[pl.ds(r0, n)]))

        plsc.subcore_barrier()

        def body(j, carry):
            off_a = base + (2 * j) * K
            off_b = off_a + K
            pltpu.sync_copy(src_hbm.at[pl.ds(off_a, K)], src_a)
            pltpu.sync_copy(dst_hbm.at[pl.ds(off_a, K)], dst_a)
            ga = pltpu.async_copy(table_hbm.at[src_a], rows_a, gsem_a)
            pltpu.sync_copy(src_hbm.at[pl.ds(off_b, K)], src_b)
            pltpu.sync_copy(dst_hbm.at[pl.ds(off_b, K)], dst_b)
            gb = pltpu.async_copy(table_hbm.at[src_b], rows_b, gsem_b)
            ga.wait()
            sa = pltpu.async_copy(rows_a, accum.at[dst_a], ssem_a, add=True)
            gb.wait()
            sb = pltpu.async_copy(rows_b, accum.at[dst_b], ssem_b, add=True)
            sa.wait()
            sb.wait()
            return carry

        lax.fori_loop(0, NFULL // 2, body, 0)
        off = base + NFULL * K
        pltpu.sync_copy(src_hbm.at[pl.ds(off, REM)], srcr_v)
        pltpu.sync_copy(dst_hbm.at[pl.ds(off, REM)], dstr_v)
        pltpu.async_copy(table_hbm.at[srcr_v], rowsr_v, gsem_a).wait()
        pltpu.sync_copy(rowsr_v, accum.at[dstr_v], add=True)
        plsc.subcore_barrier()
        _per_rows(s, lambda r0, n: pltpu.sync_copy(
            accum.at[pl.ds(r0, n)], out_hbm.at[c, pl.ds(r0, n)]))

    return agg


_agg128 = _make_agg(H1)
_agg64 = _make_agg(H2)


_BM = 2000


def _prep_body(x_ref, w1_ref, d0_ref, d1_ref, table_ref, dinv_ref):
    deg = d0_ref[:, 0] + d1_ref[:, 0] + 1.0
    dinv = lax.rsqrt(deg)
    xw = jnp.dot(x_ref[...], w1_ref[...], preferred_element_type=jnp.float32)
    table_ref[...] = xw * dinv[:, None]
    dinv_ref[...] = dinv[:, None]


_prep = pl.pallas_call(
    _prep_body,
    grid=(N // _BM,),
    in_specs=[
        pl.BlockSpec((_BM, F_IN), lambda i: (i, 0)),
        pl.BlockSpec((F_IN, H1), lambda i: (0, 0)),
        pl.BlockSpec((_BM, DEGW), lambda i: (i, 0)),
        pl.BlockSpec((_BM, DEGW), lambda i: (i, 0)),
    ],
    out_specs=[
        pl.BlockSpec((_BM, H1), lambda i: (i, 0)),
        pl.BlockSpec((_BM, 1), lambda i: (i, 0)),
    ],
    out_shape=[
        jax.ShapeDtypeStruct((N, H1), jnp.float32),
        jax.ShapeDtypeStruct((N, 1), jnp.float32),
    ],
)


def _mid_body(p0_ref, p1_ref, dinv_ref, b1_ref, w2_ref, t2_ref):
    dinv = dinv_ref[...]
    h = jnp.maximum((p0_ref[...] + p1_ref[...]) * dinv
                    + b1_ref[...][None, :], 0.0)
    t2 = jnp.dot(h, w2_ref[...], preferred_element_type=jnp.float32)
    t2_ref[...] = t2 * dinv


_mid = pl.pallas_call(
    _mid_body,
    grid=(N // _BM,),
    in_specs=[
        pl.BlockSpec((_BM, H1), lambda i: (i, 0)),
        pl.BlockSpec((_BM, H1), lambda i: (i, 0)),
        pl.BlockSpec((_BM, 1), lambda i: (i, 0)),
        pl.BlockSpec((H1,), lambda i: (0,)),
        pl.BlockSpec((H1, H2), lambda i: (0, 0)),
    ],
    out_specs=pl.BlockSpec((_BM, H2), lambda i: (i, 0)),
    out_shape=jax.ShapeDtypeStruct((N, H2), jnp.float32),
)


def _fin_body(q0_ref, q1_ref, dinv_ref, b2_ref, out_ref):
    out_ref[...] = ((q0_ref[...] + q1_ref[...]) * dinv_ref[...]
                    + b2_ref[...][None, :])


_fin = pl.pallas_call(
    _fin_body,
    grid=(N // _BM,),
    in_specs=[
        pl.BlockSpec((_BM, H2), lambda i: (i, 0)),
        pl.BlockSpec((_BM, H2), lambda i: (i, 0)),
        pl.BlockSpec((_BM, 1), lambda i: (i, 0)),
        pl.BlockSpec((H2,), lambda i: (0,)),
    ],
    out_specs=pl.BlockSpec((_BM, H2), lambda i: (i, 0)),
    out_shape=jax.ShapeDtypeStruct((N, H2), jnp.float32),
)


def kernel(x, edge_index, W1, b1, W2, b2):
    src = edge_index[0]
    dst = edge_index[1]
    ones_deg = jnp.ones((K, DEGW), jnp.float32)
    zeros_deg = jnp.zeros((RPS_A, DEGW), jnp.float32)
    zeros_h1 = jnp.zeros((RPS_A, H1), jnp.float32)
    zeros_h2 = jnp.zeros((RPS_A, H2), jnp.float32)

    deg = _deg_kernel(dst, ones_deg, zeros_deg)
    table1, dinv = _prep(x, W1, deg[0], deg[1])
    p = _agg128(src, dst, table1, zeros_h1)
    table2 = _mid(p[0], p[1], dinv, b1, W2)
    q = _agg64(src, dst, table2, zeros_h2)
    return _fin(q[0], q[1], dinv, b2)
